# baseline (speedup 1.0000x reference)
import jax
import jax.numpy as jnp
import numpy as np
from jax.experimental import pallas as pl
from jax.experimental.pallas import tpu as pltpu


def _deal_mat(r):
    m = r // 4
    d = np.zeros((r, r), np.float32)
    f = np.arange(m)
    for j in range(4):
        d[m * j + f, 4 * f + j] = 1.0
    return d


_D512 = _deal_mat(512)
_DT512 = np.ascontiguousarray(_D512.T)


def _elu(x):
    return jnp.where(x > 0, x, jnp.exp(x) - 1.0)


def _bn(x):
    mu = jnp.mean(x, axis=0, keepdims=True)
    ms = jnp.mean(x * x, axis=0, keepdims=True)
    return (x - mu) * jax.lax.rsqrt(jnp.maximum(ms - mu * mu, 0.0) + 1e-5)


def _dot(a, b):
    return jax.lax.dot_general(a, b, (((1,), (0,)), ((), ())),
                               preferred_element_type=jnp.float32)


def _to_q(x, dt):
    m = x.shape[0]
    blocks = []
    for m0 in range(0, m, 128):
        y = jnp.concatenate(
            [x[m0:m0 + 128, 32 * j:32 * (j + 1)] for j in range(4)], axis=0)
        blocks.append(_dot(dt, y))
    return jnp.concatenate(blocks, axis=0)


def _from_q(p, d):
    m = p.shape[0] // 4
    pd = _dot(d, p)
    return jnp.concatenate([pd[j * m:(j + 1) * m, :] for j in range(4)],
                           axis=1)


def _body(di_ref, da_ref, inp_ref, w1_ref, b1_ref, w0t_ref, b0_ref,
          a10_ref, a11_ref, a12_ref, a13_ref, wm_ref, bm_ref,
          w2a_ref, w2b_ref, b2_ref, a30_ref, a31_ref, a32_ref, a33_ref,
          wo_ref, bo_ref, tail_ref, d_ref, dt_ref,
          out_ref,
          dv_ref, fq_ref, daf_ref, xa_ref, v2_ref, sq_ref):
    xq_ref = fq_ref.at[0:fq_ref.shape[0] // 2]
    vh_ref = daf_ref
    i = pl.program_id(0)
    n1 = 16
    fcst = dv_ref.shape[0]
    nver = daf_ref.shape[0]

    @pl.when(i == 0)
    def _s0():
        inp = inp_ref[...]
        w1 = w1_ref[...]
        v0 = (inp[:, 0:1] * w1[0:1, :] + inp[:, 1:2] * w1[1:2, :]
              + inp[:, 2:3] * w1[2:3, :] + b1_ref[...])
        vh_ref[...] = v0
        xq_ref[...] = _to_q(_elu(v0), dt_ref[...])

    @pl.when(i == 1)
    def _s1():
        v0 = vh_ref[...]
        v2_ref[...] = v0 + _dot(_bn(_elu(v0)), w0t_ref[...]) + b0_ref[...]

    @pl.when(i == 2)
    def _s2():
        v1 = v2_ref[...]
        vh_ref[...] = _dot(_bn(_elu(v1)), a10_ref[...]) + a11_ref[...]

    @pl.when(i == 3)
    def _s3():
        v2_ref[...] += _dot(_bn(_elu(vh_ref[...])), a12_ref[...]) + a13_ref[...]

    @pl.when(i == 4)
    def _s4():
        xa_ref[...] = _dot(_bn(_elu(v2_ref[...])), w2a_ref[...])

    @pl.when(i < n1)
    def _di_step():
        blk = _from_q(_dot(di_ref[...], xq_ref[...]), d_ref[...])
        dv_ref[pl.ds(i * (fcst // n1), fcst // n1), :] = blk
        sb = jnp.sum(blk, 0, keepdims=True)
        qb = jnp.sum(blk * blk, 0, keepdims=True)

        @pl.when(i == 0)
        def _st0():
            sq_ref[0:1, :] = sb
            sq_ref[1:2, :] = qb

        @pl.when(i > 0)
        def _stn():
            sq_ref[0:1, :] += sb
            sq_ref[1:2, :] += qb

    @pl.when(i == n1)
    def _mid():
        mu = sq_ref[0:1, :] / fcst
        rs = jax.lax.rsqrt(
            jnp.maximum(sq_ref[1:2, :] / fcst - mu * mu, 0.0) + 1e-5)
        wm = wm_ref[...]
        bm = bm_ref[...]
        for c0 in range(0, fcst, 512):
            ch = dv_ref[c0:c0 + 512, :]
            dv_ref[c0:c0 + 512, :] = _elu(_dot((ch - mu) * rs, wm) + bm)
        dt = dt_ref[...]
        for m0 in range(0, fcst, 128):
            y = jnp.concatenate(
                [dv_ref[m0:m0 + 128, 32 * j:32 * (j + 1)] for j in range(4)],
                axis=0)
            fq_ref[4 * m0:4 * m0 + 512, :] = _dot(dt, y)

    @pl.when(i >= n1)
    def _da_step():
        k = i - n1
        blk = _from_q(_dot(da_ref[...], fq_ref[...]), d_ref[...])
        nb = da_ref.shape[0] // 4
        daf_ref[pl.ds(k * nb, nb), :] = blk
        sb = jnp.sum(blk, 0, keepdims=True)
        qb = jnp.sum(blk * blk, 0, keepdims=True)

        @pl.when(k == 0)
        def _st0():
            sq_ref[2:3, :] = sb
            sq_ref[3:4, :] = qb

        @pl.when(k > 0)
        def _stn():
            sq_ref[2:3, :] += sb
            sq_ref[3:4, :] += qb

    @pl.when(i == pl.num_programs(0) - 1)
    def _fin():
        mu = sq_ref[2:3, :] / nver
        rs = jax.lax.rsqrt(
            jnp.maximum(sq_ref[3:4, :] / nver - mu * mu, 0.0) + 1e-5)
        v2 = v2_ref[...]
        x = xa_ref[...] + _dot((daf_ref[...] - mu) * rs, w2b_ref[...])
        v3 = v2 + x + b2_ref[...]
        h = _dot(_bn(_elu(v3)), a30_ref[...]) + a31_ref[...]
        v4 = v3 + _dot(_bn(_elu(h)), a32_ref[...]) + a33_ref[...]
        out_ref[...] = (_dot(_bn(_elu(v4)), wo_ref[...]) + bo_ref[...]
                        + tail_ref[...])


def _const_spec(shape):
    return pl.BlockSpec(shape, lambda i: tuple(0 for _ in shape))


def kernel(Di, DiA, mask, inputs, W1, b1, rn0_W0, rn0_b0, rn0_W1, rn0_b1,
           rn1_W0, rn1_b0, rn1_W1, rn1_b1, rn2_W0, rn2_b0, rn2_W1, rn2_b1,
           rn3_W0, rn3_b0, rn3_W1, rn3_b1, W2, b2):
    del mask
    n = inputs.shape[1]
    fc = Di.shape[1] // 4
    k1 = Di.shape[2]
    k2 = DiA.shape[2]
    br = 512
    n1 = 4 * fc // br
    n2 = 4 * n // br
    inp = inputs[0]
    r1 = lambda v: v.reshape(1, -1)
    tail = jnp.tile(inp[:, -3:], (1, 40))

    out = pl.pallas_call(
        _body,
        grid=(n1 + n2,),
        in_specs=[
            pl.BlockSpec((br, k1), lambda i: (jnp.minimum(i, n1 - 1), 0)),
            pl.BlockSpec((br, k2), lambda i: (jnp.maximum(i - n1, 0), 0)),
            _const_spec((n, 3)),
            _const_spec((3, 128)),
            _const_spec((1, 128)),
            _const_spec((128, 128)),
            _const_spec((1, 128)),
            _const_spec((128, 128)),
            _const_spec((1, 128)),
            _const_spec((128, 128)),
            _const_spec((1, 128)),
            _const_spec((128, 128)),
            _const_spec((1, 128)),
            _const_spec((128, 128)),
            _const_spec((128, 128)),
            _const_spec((1, 128)),
            _const_spec((128, 128)),
            _const_spec((1, 128)),
            _const_spec((128, 128)),
            _const_spec((1, 128)),
            _const_spec((128, 120)),
            _const_spec((1, 120)),
            _const_spec((n, 120)),
            _const_spec((512, 512)),
            _const_spec((512, 512)),
        ],
        out_specs=_const_spec((n, 120)),
        out_shape=jax.ShapeDtypeStruct((n, 120), jnp.float32),
        compiler_params=pltpu.CompilerParams(
            vmem_limit_bytes=100 * 1024 * 1024),
        scratch_shapes=[
            pltpu.VMEM((fc, 128), jnp.float32),
            pltpu.VMEM((k2, 32), jnp.float32),
            pltpu.VMEM((n, 128), jnp.float32),
            pltpu.VMEM((n, 128), jnp.float32),
            pltpu.VMEM((n, 128), jnp.float32),
            pltpu.VMEM((8, 128), jnp.float32),
        ],
    )(Di[0], DiA[0], inp, W1, r1(b1), rn0_W0[:128], r1(rn0_b0),
      rn1_W0[:128], r1(rn1_b0), rn1_W1[:128], r1(rn1_b1),
      rn0_W1[128:], r1(rn0_b1),
      rn2_W0[:128], rn2_W0[128:], r1(rn2_b0),
      rn3_W0[:128], r1(rn3_b0), rn3_W1[:128], r1(rn3_b1),
      W2, r1(b2), tail,
      jnp.asarray(_D512), jnp.asarray(_DT512))
    return out.reshape(1, n, 120)


# speedup vs baseline: 1.0434x; 1.0434x over previous
import jax
import jax.numpy as jnp
import numpy as np
from jax.experimental import pallas as pl
from jax.experimental.pallas import tpu as pltpu


def _deal_mat(r):
    m = r // 4
    d = np.zeros((r, r), np.float32)
    f = np.arange(m)
    for j in range(4):
        d[m * j + f, 4 * f + j] = 1.0
    return d


_D512 = _deal_mat(512)
_DT512 = np.ascontiguousarray(_D512.T)
_D256 = _deal_mat(256)


def _elu(x):
    return jnp.where(x > 0, x, jnp.exp(x) - 1.0)


def _bn(x):
    mu = jnp.mean(x, axis=0, keepdims=True)
    ms = jnp.mean(x * x, axis=0, keepdims=True)
    return (x - mu) * jax.lax.rsqrt(jnp.maximum(ms - mu * mu, 0.0) + 1e-5)


def _dot(a, b):
    return jax.lax.dot_general(a, b, (((1,), (0,)), ((), ())),
                               preferred_element_type=jnp.float32)


def _avg_block(v, w0, b0, w1, b1):
    h = _dot(_bn(_elu(v)), w0) + b0
    return v + _dot(_bn(_elu(h)), w1) + b1


def _to_q(x, dt):
    m = x.shape[0]
    blocks = []
    for m0 in range(0, m, 128):
        y = jnp.concatenate(
            [x[m0:m0 + 128, 32 * j:32 * (j + 1)] for j in range(4)], axis=0)
        blocks.append(_dot(dt, y))
    return jnp.concatenate(blocks, axis=0)


def _from_q(p, d):
    r = d.shape[0]
    m = r // 4
    outs = []
    for p0 in range(0, p.shape[0], r):
        pd = _dot(d, p[p0:p0 + r, :])
        outs.append(jnp.concatenate(
            [pd[j * m:(j + 1) * m, :] for j in range(4)], axis=1))
    return jnp.concatenate(outs, axis=0) if len(outs) > 1 else outs[0]


def _k1_body(di_ref, inp_ref, w1_ref, b1_ref, w0t_ref, b0_ref,
             a0_ref, a1_ref, a2_ref, a3_ref, d_ref, dt_ref,
             dv_ref, v2_ref, sqo_ref, xq_ref, vh_ref):
    i = pl.program_id(0)

    @pl.when(i == 0)
    def _s0():
        inp = inp_ref[...]
        w1 = w1_ref[...]
        v0 = (inp[:, 0:1] * w1[0:1, :] + inp[:, 1:2] * w1[1:2, :]
              + inp[:, 2:3] * w1[2:3, :] + b1_ref[...])
        vh_ref[...] = v0
        xq_ref[...] = _to_q(_elu(v0), dt_ref[...])

    @pl.when(i == 1)
    def _s1():
        v0 = vh_ref[...]
        v2_ref[...] = v0 + _dot(_bn(_elu(v0)), w0t_ref[...]) + b0_ref[...]

    @pl.when(i == 2)
    def _s2():
        v1 = v2_ref[...]
        vh_ref[...] = _dot(_bn(_elu(v1)), a0_ref[...]) + a1_ref[...]

    @pl.when(i == 3)
    def _s3():
        v2_ref[...] += _dot(_bn(_elu(vh_ref[...])), a2_ref[...]) + a3_ref[...]

    blk = _from_q(_dot(di_ref[...], xq_ref[...]), d_ref[...])
    dv_ref[...] = blk
    sb = jnp.sum(blk, 0, keepdims=True)
    qb = jnp.sum(blk * blk, 0, keepdims=True)

    @pl.when(i == 0)
    def _st0():
        sqo_ref[0:1, :] = sb
        sqo_ref[1:2, :] = qb

    @pl.when(i > 0)
    def _stn():
        sqo_ref[0:1, :] += sb
        sqo_ref[1:2, :] += qb


def _k2_body(da_ref, dv_ref, wm_ref, bm_ref, v2_ref, w2a_ref, w2b_ref,
             b2_ref, a0_ref, a1_ref, a2_ref, a3_ref, wo_ref, bo_ref,
             tail_ref, d_ref, dt_ref, sq1_ref, out_ref, fq_ref, daf_ref,
             xa_ref, fqn_ref, sq_ref):
    i = pl.program_id(0)

    fcst = dv_ref.shape[0]

    @pl.when(i == 0)
    def _init():
        mu = sq1_ref[0:1, :] / fcst
        rs = jax.lax.rsqrt(
            jnp.maximum(sq1_ref[1:2, :] / fcst - mu * mu, 0.0) + 1e-5)
        wm = wm_ref[...]
        bm = bm_ref[...]
        for c0 in range(0, fcst, 512):
            ch = dv_ref[c0:c0 + 512, :]
            fqn_ref[c0:c0 + 512, :] = _elu(_dot((ch - mu) * rs, wm) + bm)
        dt = dt_ref[...]
        for m0 in range(0, fcst, 128):
            y = jnp.concatenate(
                [fqn_ref[m0:m0 + 128, 32 * j:32 * (j + 1)] for j in range(4)],
                axis=0)
            fq_ref[4 * m0:4 * m0 + 512, :] = _dot(dt, y)

    @pl.when(i == 1)
    def _s1():
        xa_ref[...] = _dot(_bn(_elu(v2_ref[...])), w2a_ref[...])

    nb = da_ref.shape[0] // 4
    blk = _from_q(_dot(da_ref[...], fq_ref[...]), d_ref[...])
    daf_ref[pl.ds(i * nb, nb), :] = blk
    sb = jnp.sum(blk, 0, keepdims=True)
    qb = jnp.sum(blk * blk, 0, keepdims=True)

    @pl.when(i == 0)
    def _st0():
        sq_ref[0:1, :] = sb
        sq_ref[1:2, :] = qb

    @pl.when(i > 0)
    def _stn():
        sq_ref[0:1, :] += sb
        sq_ref[1:2, :] += qb

    @pl.when(i == pl.num_programs(0) - 1)
    def _fin():
        nrows = daf_ref.shape[0]
        mu = sq_ref[0:1, :] / nrows
        rs = jax.lax.rsqrt(
            jnp.maximum(sq_ref[1:2, :] / nrows - mu * mu, 0.0) + 1e-5)
        v2 = v2_ref[...]
        x = xa_ref[...] + _dot((daf_ref[...] - mu) * rs, w2b_ref[...])
        v3 = v2 + x + b2_ref[...]
        v4 = _avg_block(v3, a0_ref[...], a1_ref[...], a2_ref[...], a3_ref[...])
        out_ref[...] = (_dot(_bn(_elu(v4)), wo_ref[...]) + bo_ref[...]
                        + tail_ref[...])


def _const_spec(shape):
    return pl.BlockSpec(shape, lambda i: tuple(0 for _ in shape))


def kernel(Di, DiA, mask, inputs, W1, b1, rn0_W0, rn0_b0, rn0_W1, rn0_b1,
           rn1_W0, rn1_b0, rn1_W1, rn1_b1, rn2_W0, rn2_b0, rn2_W1, rn2_b1,
           rn3_W0, rn3_b0, rn3_W1, rn3_b1, W2, b2):
    del mask
    n = inputs.shape[1]
    fc = Di.shape[1] // 4
    k1 = Di.shape[2]
    k2 = DiA.shape[2]
    br = 1024
    br2 = 512
    inp = inputs[0]
    r1 = lambda v: v.reshape(1, -1)

    dv, v2, sq1 = pl.pallas_call(
        _k1_body,
        grid=(4 * fc // br,),
        in_specs=[
            pl.BlockSpec((br, k1), lambda i: (i, 0)),
            _const_spec((n, 3)),
            _const_spec((3, 128)),
            _const_spec((1, 128)),
            _const_spec((128, 128)),
            _const_spec((1, 128)),
            _const_spec((128, 128)),
            _const_spec((1, 128)),
            _const_spec((128, 128)),
            _const_spec((1, 128)),
            _const_spec((512, 512)),
            _const_spec((512, 512)),
        ],
        out_specs=[
            pl.BlockSpec((br // 4, 128), lambda i: (i, 0)),
            _const_spec((n, 128)),
            _const_spec((8, 128)),
        ],
        out_shape=[
            jax.ShapeDtypeStruct((fc, 128), jnp.float32),
            jax.ShapeDtypeStruct((n, 128), jnp.float32),
            jax.ShapeDtypeStruct((8, 128), jnp.float32),
        ],
        scratch_shapes=[pltpu.VMEM((k1, 32), jnp.float32),
                        pltpu.VMEM((n, 128), jnp.float32)],
    )(Di[0], inp, W1, r1(b1), rn0_W0[:128], r1(rn0_b0),
      rn1_W0[:128], r1(rn1_b0), rn1_W1[:128], r1(rn1_b1),
      jnp.asarray(_D512), jnp.asarray(_DT512))

    tail = jnp.tile(inp[:, -3:], (1, 40))
    out = pl.pallas_call(
        _k2_body,
        grid=(4 * n // br2,),
        in_specs=[
            pl.BlockSpec((br2, k2), lambda i: (i, 0)),
            _const_spec((fc, 128)),
            _const_spec((128, 128)),
            _const_spec((1, 128)),
            _const_spec((n, 128)),
            _const_spec((128, 128)),
            _const_spec((128, 128)),
            _const_spec((1, 128)),
            _const_spec((128, 128)),
            _const_spec((1, 128)),
            _const_spec((128, 128)),
            _const_spec((1, 128)),
            _const_spec((128, 120)),
            _const_spec((1, 120)),
            _const_spec((n, 120)),
            _const_spec((256, 256)),
            _const_spec((512, 512)),
            _const_spec((8, 128)),
        ],
        out_specs=_const_spec((n, 120)),
        out_shape=jax.ShapeDtypeStruct((n, 120), jnp.float32),
        compiler_params=pltpu.CompilerParams(
            vmem_limit_bytes=100 * 1024 * 1024),
        scratch_shapes=[pltpu.VMEM((k2, 32), jnp.float32),
                        pltpu.VMEM((n, 128), jnp.float32),
                        pltpu.VMEM((n, 128), jnp.float32),
                        pltpu.VMEM((fc, 128), jnp.float32),
                        pltpu.VMEM((8, 128), jnp.float32)],
    )(DiA[0], dv, rn0_W1[128:], r1(rn0_b1), v2,
      rn2_W0[:128], rn2_W0[128:], r1(rn2_b0),
      rn3_W0[:128], r1(rn3_b0), rn3_W1[:128], r1(rn3_b1),
      W2, r1(b2), tail, jnp.asarray(_D256), jnp.asarray(_DT512), sq1)
    return out.reshape(1, n, 120)
